# baseline (speedup 1.0000x reference)
import jax
import jax.numpy as jnp
import numpy as np
from jax.experimental import pallas as pl

_B_TILE = 32
_CH = 4
_T = 32
_NT = 24
_D = 256
_H = 8
_DH = _D // _H
_NA = 19
_NL = 2
_NEG = -1e9
_BF = jnp.bfloat16


def _dot_t(a, b):
    return jax.lax.dot_general(a, b, (((1,), (1,)), ((), ())),
                               preferred_element_type=jnp.float32)


def _dot(a, b):
    return jax.lax.dot_general(a, b, (((1,), (0,)), ((), ())),
                               preferred_element_type=jnp.float32)


def _bdot(a, w_ref):
    return _dot(a.astype(_BF), w_ref[...])


def _body(g_ref, hand_ref, jok_ref, cons_ref, shop_ref, pack_ref,
          em_ref, tm_ref, cm_ref, pm_ref, *rest):
    n_par = 12 + _NL * 11 + 8
    par = rest[:n_par]
    type_out, ptr_out, card_out, val_out = rest[n_par:]

    it = iter(par)
    W_hand, b_hand, W_joker, b_joker, W_cons, b_cons, W_shop, b_shop, \
        W_pack, b_pack, W_glob, b_glob = (next(it) for _ in range(12))
    layers = []
    for _ in range(_NL):
        layers.append(tuple(next(it) for _ in range(11)))
    lnf_g, lnf_b, W_type, b_type, A_act, Wq_ptr, Wk_ptr, W_cv = \
        (next(it) for _ in range(8))

    BT = _B_TILE
    R = BT * _T
    RC = _CH * _T
    NCH = BT // _CH

    ones_d = jnp.ones((_D, 1), jnp.float32)
    inv_d = np.float32(1.0 / _D)

    def lnmm(x, g, b):
        m = _dot(x, ones_d) * inv_d
        xc = x - m
        v = _dot(xc * xc, ones_d) * inv_d
        return xc * jax.lax.rsqrt(v + 1e-5) * g[...] + b[...]

    g = _dot(g_ref[...], W_glob[...]) + b_glob[...]
    h_hand = _dot(hand_ref[...].reshape(BT * 8, 32), W_hand[...]) + b_hand[...]
    h_jok = _dot(jok_ref[...].reshape(BT * 5, 64), W_joker[...]) + b_joker[...]
    h_cons = _dot(cons_ref[...].reshape(BT * 2, 32), W_cons[...]) + b_cons[...]
    h_shop = _dot(shop_ref[...].reshape(BT * 4, 64), W_shop[...]) + b_shop[...]
    h_pack = _dot(pack_ref[...].reshape(BT * 5, 32), W_pack[...]) + b_pack[...]

    x = jnp.concatenate([
        g.reshape(BT, 1, _D),
        h_hand.reshape(BT, 8, _D),
        h_jok.reshape(BT, 5, _D),
        h_cons.reshape(BT, 2, _D),
        h_shop.reshape(BT, 4, _D),
        h_pack.reshape(BT, 5, _D),
        jnp.zeros((BT, _T - 1 - _NT, _D), jnp.float32),
    ], axis=1)
    x2 = x.reshape(R, _D)

    em = em_ref[...]
    keyf = jnp.concatenate([
        jnp.ones((BT, 1), jnp.float32), em,
        jnp.zeros((BT, _T - 1 - _NT), jnp.float32)], axis=1)
    kv_rows = jnp.broadcast_to(keyf[:, None, :], (BT, _T, _T)).reshape(R, _T)
    rc_rows = jax.lax.broadcasted_iota(jnp.int32, (RC, RC), 0) // _T
    rc_cols = jax.lax.broadcasted_iota(jnp.int32, (RC, RC), 1) // _T
    bd01 = (rc_rows == rc_cols).astype(jnp.float32)
    mask01 = []
    for c in range(NCH):
        kv_c = kv_rows[c * RC:(c + 1) * RC]
        mask01.append(bd01 * jnp.concatenate([kv_c] * _CH, axis=1))

    inv_sqrt_dh = np.float32(1.0 / np.sqrt(_DH))
    ones_rc = jnp.ones((RC, 1), _BF)

    for (ln1_g, ln1_b, Wq, Wk, Wv, Wo, ln2_g, ln2_b, W1, b1, W2) in layers:
        h = lnmm(x2, ln1_g, ln1_b)
        q = (_bdot(h, Wq) * inv_sqrt_dh).astype(_BF)
        k = _bdot(h, Wk).astype(_BF)
        v = _bdot(h, Wv).astype(_BF)
        o_chunks = []
        for c in range(NCH):
            rs = slice(c * RC, (c + 1) * RC)
            qc, kc, vc = q[rs], k[rs], v[rs]
            heads = []
            for hd in range(_H):
                sl = slice(hd * _DH, (hd + 1) * _DH)
                e = jnp.exp(_dot_t(qc[:, sl], kc[:, sl])) * mask01[c]
                vaug = jnp.concatenate([vc[:, sl], ones_rc], axis=1)
                eo = _dot(e.astype(_BF), vaug)
                heads.append(eo[:, :_DH] / eo[:, _DH:_DH + 1])
            o_chunks.append(jnp.concatenate(heads, axis=1))
        o = jnp.concatenate(o_chunks, axis=0)
        x2 = x2 + _bdot(o, Wo)
        h = lnmm(x2, ln2_g, ln2_b)
        x2 = x2 + _dot(jax.nn.gelu(_bdot(h, W1) + b1[...]).astype(_BF),
                       W2[...])

    x2 = lnmm(x2, lnf_g, lnf_b)
    x3 = x2.reshape(BT, _T, _D)
    h0 = x3[:, 0, :]

    tl = _bdot(h0, W_type) + b_type[...]
    type_out[...] = jnp.where(tm_ref[...] > 0.5, tl, _NEG)

    qp = _bdot(h0, Wq_ptr)
    kp = _bdot(x2, Wk_ptr)
    t2 = _dot_t(kp.astype(_BF), A_act[...])
    qp_rows = jnp.broadcast_to(qp[:, None, :], (BT, _T, _D)).reshape(R, _D)
    t1 = _dot(kp * qp_rows, ones_d)
    inv_sqrt_d = np.float32(1.0 / np.sqrt(_D))
    ptr_flat = (t2 + t1) * inv_sqrt_d

    diag01 = (jax.lax.broadcasted_iota(jnp.int32, (R, _T), 1)
              == jax.lax.broadcasted_iota(jnp.int32, (R, _T), 0) % _T)
    kv_diag = jnp.sum(kv_rows * diag01.astype(jnp.float32),
                      axis=1, keepdims=True)
    pm_flat = pm_ref[...].reshape(R, _NA)
    ptr_m = jnp.where((pm_flat > 0.5) & (kv_diag > 0.5), ptr_flat, _NEG)
    ptr_out[...] = ptr_m.reshape(BT, _T, _NA)[:, 1:1 + _NT, :]

    cv = _bdot(x2, W_cv)
    cv3 = cv.reshape(BT, _T, 2)
    card_out[...] = jnp.where(cm_ref[...] > 0.5, cv3[:, 1:9, 0:1], _NEG)
    val_out[...] = cv3[:, 0:1, 1:2]


@jax.jit
def kernel(global_context, hand_cards, jokers, consumables, shop_cards,
           pack_cards, hand_mask, joker_mask, cons_mask, shop_mask, pack_mask,
           type_mask, card_mask, pointer_masks, params):
    p = params
    B = global_context.shape[0]
    BT = _B_TILE
    NB = B // BT

    ent_maskf = jnp.concatenate(
        [hand_mask, joker_mask, cons_mask, shop_mask, pack_mask],
        axis=1).astype(jnp.float32)
    type_maskf = type_mask.astype(jnp.float32)
    card_maskf = card_mask.astype(jnp.float32).reshape(B, 8, 1)
    pm_pad = jnp.pad(pointer_masks.astype(jnp.float32).transpose(0, 2, 1),
                     ((0, 0), (1, _T - 1 - _NT), (0, 0)))

    def v2(w):
        return w.reshape(1, -1)

    def bf(w):
        return w.astype(_BF)

    par = [bf(p['W_hand']), v2(p['b_hand']), bf(p['W_joker']), v2(p['b_joker']),
           bf(p['W_cons']), v2(p['b_cons']), bf(p['W_shop']), v2(p['b_shop']),
           bf(p['W_pack']), v2(p['b_pack']), bf(p['W_glob']), v2(p['b_glob'])]
    for i in range(_NL):
        par += [v2(p[f'l{i}_ln1_g']), v2(p[f'l{i}_ln1_b']),
                bf(p[f'l{i}_Wq']), bf(p[f'l{i}_Wk']), bf(p[f'l{i}_Wv']),
                bf(p[f'l{i}_Wo']),
                v2(p[f'l{i}_ln2_g']), v2(p[f'l{i}_ln2_b']),
                bf(p[f'l{i}_W1']), v2(p[f'l{i}_b1']), bf(p[f'l{i}_W2'])]
    par += [v2(p['lnf_g']), v2(p['lnf_b']), bf(p['W_type']), v2(p['b_type']),
            bf(p['A_act']), bf(p['Wq_ptr']), bf(p['Wk_ptr']),
            bf(jnp.stack([p['w_card'], p['w_val']], axis=1))]

    def tile_spec(shape):
        blk = (BT,) + shape[1:]
        nz = (0,) * (len(shape) - 1)
        return pl.BlockSpec(blk, lambda i, _nz=nz: (i,) + _nz)

    def full_spec(shape):
        nz = (0,) * len(shape)
        return pl.BlockSpec(shape, lambda i, _nz=nz: _nz)

    data = [bf(global_context), bf(hand_cards), bf(jokers), bf(consumables),
            bf(shop_cards), bf(pack_cards), ent_maskf, type_maskf,
            card_maskf, pm_pad]
    in_specs = [tile_spec(a.shape) for a in data] + \
               [full_spec(w.shape) for w in par]

    out_shape = [
        jax.ShapeDtypeStruct((B, _NA), jnp.float32),
        jax.ShapeDtypeStruct((B, _NT, _NA), jnp.float32),
        jax.ShapeDtypeStruct((B, 8, 1), jnp.float32),
        jax.ShapeDtypeStruct((B, 1, 1), jnp.float32),
    ]
    out_specs = [tile_spec(s.shape) for s in out_shape]

    type_logits, ptr_raw, card_raw, val_raw = pl.pallas_call(
        _body,
        grid=(NB,),
        in_specs=in_specs,
        out_specs=out_specs,
        out_shape=out_shape,
    )(*data, *par)

    return (type_logits, ptr_raw.transpose(0, 2, 1),
            card_raw.reshape(B, 8), val_raw.reshape(B))


# speedup vs baseline: 1.9420x; 1.9420x over previous
import jax
import jax.numpy as jnp
import numpy as np
from jax.experimental import pallas as pl

_B_TILE = 32
_CH = 8
_T = 32
_NT = 24
_D = 256
_H = 8
_DH = _D // _H
_NA = 19
_NL = 2
_NEG = -1e9


def _dot_t(a, b):
    return jax.lax.dot_general(a, b, (((1,), (1,)), ((), ())),
                               preferred_element_type=jnp.float32)


def _dot(a, b):
    return jax.lax.dot_general(a, b, (((1,), (0,)), ((), ())),
                               preferred_element_type=jnp.float32)


def _body(g_ref, hand_ref, jok_ref, cons_ref, shop_ref, pack_ref,
          em_ref, tm_ref, cm_ref, pm_ref, *rest):
    n_par = 12 + _NL * 11 + 8
    par = rest[:n_par]
    type_out, ptr_out, card_out, val_out = rest[n_par:]

    it = iter(par)
    W_hand, b_hand, W_joker, b_joker, W_cons, b_cons, W_shop, b_shop, \
        W_pack, b_pack, W_glob, b_glob = (next(it) for _ in range(12))
    layers = []
    for _ in range(_NL):
        layers.append(tuple(next(it) for _ in range(11)))
    lnf_g, lnf_b, W_type, b_type, A_act, Wq_ptr, Wk_ptr, W_cv = \
        (next(it) for _ in range(8))

    BT = _B_TILE
    R = BT * _T
    RC = _CH * _T
    NCH = BT // _CH

    ones_d = jnp.ones((_D, 1), jnp.float32)
    inv_d = np.float32(1.0 / _D)

    def lnmm(x, g, b):
        m = _dot(x, ones_d) * inv_d
        xc = x - m
        v = _dot(xc * xc, ones_d) * inv_d
        return xc * jax.lax.rsqrt(v + 1e-5) * g[...] + b[...]

    g = _dot(g_ref[...], W_glob[...]) + b_glob[...]
    h_hand = _dot(hand_ref[...].reshape(BT * 8, 32), W_hand[...]) + b_hand[...]
    h_jok = _dot(jok_ref[...].reshape(BT * 5, 64), W_joker[...]) + b_joker[...]
    h_cons = _dot(cons_ref[...].reshape(BT * 2, 32), W_cons[...]) + b_cons[...]
    h_shop = _dot(shop_ref[...].reshape(BT * 4, 64), W_shop[...]) + b_shop[...]
    h_pack = _dot(pack_ref[...].reshape(BT * 5, 32), W_pack[...]) + b_pack[...]

    x = jnp.concatenate([
        g.reshape(BT, 1, _D),
        h_hand.reshape(BT, 8, _D),
        h_jok.reshape(BT, 5, _D),
        h_cons.reshape(BT, 2, _D),
        h_shop.reshape(BT, 4, _D),
        h_pack.reshape(BT, 5, _D),
        jnp.zeros((BT, _T - 1 - _NT, _D), jnp.float32),
    ], axis=1)
    x2 = x.reshape(R, _D)

    em = em_ref[...]
    keyf = jnp.concatenate([
        jnp.ones((BT, 1), jnp.float32), em,
        jnp.zeros((BT, _T - 1 - _NT), jnp.float32)], axis=1)
    kv_rows = jnp.broadcast_to(keyf[:, None, :], (BT, _T, _T)).reshape(R, _T)
    diag01 = (jax.lax.broadcasted_iota(jnp.int32, (R, _T), 1)
              == jax.lax.broadcasted_iota(jnp.int32, (R, _T), 0) % _T)
    kv_diag = jnp.sum(kv_rows * diag01.astype(jnp.float32),
                      axis=1, keepdims=True)
    kvcol = kv_diag * 100.0 - 200.0
    z10 = 10.0 * (jax.lax.broadcasted_iota(jnp.int32, (RC, _CH), 0) // _T
                  == jax.lax.broadcasted_iota(jnp.int32, (RC, _CH), 1)
                  ).astype(jnp.float32)

    inv_sqrt_dh = np.float32(1.0 / np.sqrt(_DH))
    ones_rc = jnp.ones((RC, 1), jnp.float32)
    ones_bc = jnp.ones((RC, _DH), jnp.float32)
    sqrt_half = np.float32(1.0 / np.sqrt(2.0))

    for (ln1_g, ln1_b, Wq, Wk, Wv, Wo, ln2_g, ln2_b, W1, b1, W2) in layers:
        h = lnmm(x2, ln1_g, ln1_b)
        q = _dot(h, Wq[...]) * inv_sqrt_dh
        k = _dot(h, Wk[...])
        v = _dot(h, Wv[...])
        o_chunks = []
        for c in range(NCH):
            rs = slice(c * RC, (c + 1) * RC)
            qc, kc, vc = q[rs], k[rs], v[rs]
            kvc = kvcol[rs]
            heads = []
            for hd in range(_H):
                sl = slice(hd * _DH, (hd + 1) * _DH)
                q_aug = jnp.concatenate([qc[:, sl], z10, ones_rc], axis=1)
                k_aug = jnp.concatenate([kc[:, sl], z10, kvc], axis=1)
                e = jnp.exp(_dot_t(q_aug, k_aug))
                vaug = jnp.concatenate([vc[:, sl], ones_bc], axis=1)
                eo = _dot(e, vaug)
                heads.append(eo[:, :_DH]
                             * jax.lax.reciprocal(eo[:, _DH:]))
            o_chunks.append(jnp.concatenate(heads, axis=1))
        o = jnp.concatenate(o_chunks, axis=0)
        x2 = x2 + _dot(o, Wo[...])
        h = lnmm(x2, ln2_g, ln2_b)
        ff = _dot(h, W1[...]) + b1[...]
        ff = ff * 0.5 * (1.0 + jax.lax.erf(ff * sqrt_half))
        x2 = x2 + _dot(ff, W2[...])

    x2 = lnmm(x2, lnf_g, lnf_b)
    x3 = x2.reshape(BT, _T, _D)
    h0 = x3[:, 0, :]

    tl = _dot(h0, W_type[...]) + b_type[...]
    type_out[...] = jnp.where(tm_ref[...] > 0.5, tl, _NEG)

    qp = _dot(h0, Wq_ptr[...])
    kp = _dot(x2, Wk_ptr[...])
    t2 = _dot_t(kp, A_act[...])
    qp_rows = jnp.broadcast_to(qp[:, None, :], (BT, _T, _D)).reshape(R, _D)
    t1 = _dot(kp * qp_rows, ones_d)
    inv_sqrt_d = np.float32(1.0 / np.sqrt(_D))
    ptr_flat = (t2 + t1) * inv_sqrt_d

    pm_flat = pm_ref[...].reshape(R, _NA)
    ptr_m = jnp.where((pm_flat > 0.5) & (kv_diag > 0.5), ptr_flat, _NEG)
    ptr_out[...] = ptr_m.reshape(BT, _T, _NA)[:, 1:1 + _NT, :]

    cv = _dot(x2, W_cv[...])
    cv3 = cv.reshape(BT, _T, 2)
    card_out[...] = jnp.where(cm_ref[...] > 0.5, cv3[:, 1:9, 0:1], _NEG)
    val_out[...] = cv3[:, 0:1, 1:2]


@jax.jit
def kernel(global_context, hand_cards, jokers, consumables, shop_cards,
           pack_cards, hand_mask, joker_mask, cons_mask, shop_mask, pack_mask,
           type_mask, card_mask, pointer_masks, params):
    p = params
    B = global_context.shape[0]
    BT = _B_TILE
    NB = B // BT

    ent_maskf = jnp.concatenate(
        [hand_mask, joker_mask, cons_mask, shop_mask, pack_mask],
        axis=1).astype(jnp.float32)
    type_maskf = type_mask.astype(jnp.float32)
    card_maskf = card_mask.astype(jnp.float32).reshape(B, 8, 1)
    pm_pad = jnp.pad(pointer_masks.astype(jnp.float32).transpose(0, 2, 1),
                     ((0, 0), (1, _T - 1 - _NT), (0, 0)))

    def v2(w):
        return w.reshape(1, -1)

    par = [p['W_hand'], v2(p['b_hand']), p['W_joker'], v2(p['b_joker']),
           p['W_cons'], v2(p['b_cons']), p['W_shop'], v2(p['b_shop']),
           p['W_pack'], v2(p['b_pack']), p['W_glob'], v2(p['b_glob'])]
    for i in range(_NL):
        par += [v2(p[f'l{i}_ln1_g']), v2(p[f'l{i}_ln1_b']),
                p[f'l{i}_Wq'], p[f'l{i}_Wk'], p[f'l{i}_Wv'], p[f'l{i}_Wo'],
                v2(p[f'l{i}_ln2_g']), v2(p[f'l{i}_ln2_b']),
                p[f'l{i}_W1'], v2(p[f'l{i}_b1']), p[f'l{i}_W2']]
    par += [v2(p['lnf_g']), v2(p['lnf_b']), p['W_type'], v2(p['b_type']),
            p['A_act'], p['Wq_ptr'], p['Wk_ptr'],
            jnp.stack([p['w_card'], p['w_val']], axis=1)]

    def tile_spec(shape):
        blk = (BT,) + shape[1:]
        nz = (0,) * (len(shape) - 1)
        return pl.BlockSpec(blk, lambda i, _nz=nz: (i,) + _nz)

    def full_spec(shape):
        nz = (0,) * len(shape)
        return pl.BlockSpec(shape, lambda i, _nz=nz: _nz)

    data = [global_context, hand_cards, jokers, consumables, shop_cards,
            pack_cards, ent_maskf, type_maskf, card_maskf, pm_pad]
    in_specs = [tile_spec(a.shape) for a in data] + \
               [full_spec(w.shape) for w in par]

    out_shape = [
        jax.ShapeDtypeStruct((B, _NA), jnp.float32),
        jax.ShapeDtypeStruct((B, _NT, _NA), jnp.float32),
        jax.ShapeDtypeStruct((B, 8, 1), jnp.float32),
        jax.ShapeDtypeStruct((B, 1, 1), jnp.float32),
    ]
    out_specs = [tile_spec(s.shape) for s in out_shape]

    type_logits, ptr_raw, card_raw, val_raw = pl.pallas_call(
        _body,
        grid=(NB,),
        in_specs=in_specs,
        out_specs=out_specs,
        out_shape=out_shape,
    )(*data, *par)

    return (type_logits, ptr_raw.transpose(0, 2, 1),
            card_raw.reshape(B, 8), val_raw.reshape(B))


# speedup vs baseline: 1.9697x; 1.0142x over previous
import jax
import jax.numpy as jnp
import numpy as np
from jax.experimental import pallas as pl

_B_TILE = 32
_S_TILE = 512
_CH = 8
_T = 32
_NT = 24
_D = 256
_H = 8
_DH = _D // _H
_NA = 19
_NL = 2
_NEG = -1e9
_BF = jnp.bfloat16


def _dot_t(a, b):
    return jax.lax.dot_general(a, b, (((1,), (1,)), ((), ())),
                               preferred_element_type=jnp.float32)


def _dot(a, b):
    return jax.lax.dot_general(a, b, (((1,), (0,)), ((), ())),
                               preferred_element_type=jnp.float32)


def _body(g_ref, hand_ref, jok_ref, cons_ref, shop_ref, pack_ref,
          em_ref, tm_ref, cm_ref, pm_ref, *rest):
    n_par = 12 + _NL * 11 + 8
    par = rest[:n_par]
    type_out, ptr_out, card_out, val_out = rest[n_par:]

    it = iter(par)
    W_hand, b_hand, W_joker, b_joker, W_cons, b_cons, W_shop, b_shop, \
        W_pack, b_pack, W_glob, b_glob = (next(it) for _ in range(12))
    layers = []
    for _ in range(_NL):
        layers.append(tuple(next(it) for _ in range(11)))
    lnf_g, lnf_b, W_type, b_type, A_act, Wq_ptr, Wk_ptr, W_cv = \
        (next(it) for _ in range(8))

    BT = _B_TILE
    R = BT * _T
    RC = _CH * _T
    NCH = BT // _CH
    NB = _S_TILE // BT

    ones_d = jnp.ones((_D, 1), jnp.float32)
    inv_d = np.float32(1.0 / _D)
    inv_sqrt_dh = np.float32(1.0 / np.sqrt(_DH))
    inv_sqrt_d = np.float32(1.0 / np.sqrt(_D))
    ones_rc = jnp.ones((RC, 1), jnp.float32)
    ones_bc = jnp.ones((RC, _DH), jnp.float32)
    sqrt_half = np.float32(1.0 / np.sqrt(2.0))
    z10 = 10.0 * (jax.lax.broadcasted_iota(jnp.int32, (RC, _CH), 0) // _T
                  == jax.lax.broadcasted_iota(jnp.int32, (RC, _CH), 1)
                  ).astype(jnp.float32)
    diag01 = (jax.lax.broadcasted_iota(jnp.int32, (R, _T), 1)
              == jax.lax.broadcasted_iota(jnp.int32, (R, _T), 0) % _T
              ).astype(jnp.float32)

    def lnmm(x, g, b):
        m = _dot(x, ones_d) * inv_d
        xc = x - m
        v = _dot(xc * xc, ones_d) * inv_d
        return xc * jax.lax.rsqrt(v + 1e-5) * g[...] + b[...]

    def tile(i, carry):
        bs = pl.ds(i * BT, BT)

        f32 = jnp.float32
        g = _dot(g_ref[bs, :].astype(f32), W_glob[...]) + b_glob[...]
        h_hand = _dot(hand_ref[bs].reshape(BT * 8, 32).astype(f32),
                      W_hand[...]) + b_hand[...]
        h_jok = _dot(jok_ref[bs].reshape(BT * 5, 64).astype(f32),
                     W_joker[...]) + b_joker[...]
        h_cons = _dot(cons_ref[bs].reshape(BT * 2, 32).astype(f32),
                      W_cons[...]) + b_cons[...]
        h_shop = _dot(shop_ref[bs].reshape(BT * 4, 64).astype(f32),
                      W_shop[...]) + b_shop[...]
        h_pack = _dot(pack_ref[bs].reshape(BT * 5, 32).astype(f32),
                      W_pack[...]) + b_pack[...]

        x = jnp.concatenate([
            g.reshape(BT, 1, _D),
            h_hand.reshape(BT, 8, _D),
            h_jok.reshape(BT, 5, _D),
            h_cons.reshape(BT, 2, _D),
            h_shop.reshape(BT, 4, _D),
            h_pack.reshape(BT, 5, _D),
            jnp.zeros((BT, _T - 1 - _NT, _D), jnp.float32),
        ], axis=1)
        x2 = x.reshape(R, _D)

        em = em_ref[bs, :].astype(jnp.float32)
        keyf = jnp.concatenate([
            jnp.ones((BT, 1), jnp.float32), em,
            jnp.zeros((BT, _T - 1 - _NT), jnp.float32)], axis=1)
        kv_rows = jnp.broadcast_to(
            keyf[:, None, :], (BT, _T, _T)).reshape(R, _T)
        kv_diag = jnp.sum(kv_rows * diag01, axis=1, keepdims=True)
        kvcol = kv_diag * 100.0 - 200.0

        for (ln1_g, ln1_b, Wq, Wk, Wv, Wo, ln2_g, ln2_b, W1, b1, W2) \
                in layers:
            h = lnmm(x2, ln1_g, ln1_b)
            q = _dot(h, Wq[...]) * inv_sqrt_dh
            k = _dot(h, Wk[...])
            v = _dot(h, Wv[...])
            o_chunks = []
            for c in range(NCH):
                rs = slice(c * RC, (c + 1) * RC)
                qc, kc, vc = q[rs], k[rs], v[rs]
                kvc = kvcol[rs]
                heads = []
                for hd in range(_H):
                    sl = slice(hd * _DH, (hd + 1) * _DH)
                    q_aug = jnp.concatenate(
                        [qc[:, sl], z10, ones_rc], axis=1)
                    k_aug = jnp.concatenate([kc[:, sl], z10, kvc], axis=1)
                    e = jnp.exp(_dot_t(q_aug, k_aug))
                    vaug = jnp.concatenate([vc[:, sl], ones_bc], axis=1)
                    eo = _dot(e, vaug)
                    heads.append(eo[:, :_DH]
                                 * jax.lax.reciprocal(eo[:, _DH:]))
                o_chunks.append(jnp.concatenate(heads, axis=1))
            o = jnp.concatenate(o_chunks, axis=0)
            x2 = x2 + _dot(o, Wo[...])
            h = lnmm(x2, ln2_g, ln2_b)
            ff = _dot(h, W1[...]) + b1[...]
            ff = ff * 0.5 * (1.0 + jax.lax.erf(ff * sqrt_half))
            x2 = x2 + _dot(ff, W2[...])

        x2 = lnmm(x2, lnf_g, lnf_b)
        x3 = x2.reshape(BT, _T, _D)
        h0 = x3[:, 0, :]

        tl = _dot(h0, W_type[...]) + b_type[...]
        type_out[bs, :] = jnp.where(tm_ref[bs, :] > 0.5, tl, _NEG)

        qp = _dot(h0, Wq_ptr[...])
        kp = _dot(x2, Wk_ptr[...])
        t2 = _dot_t(kp, A_act[...])
        qp_rows = jnp.broadcast_to(
            qp[:, None, :], (BT, _T, _D)).reshape(R, _D)
        t1 = _dot(kp * qp_rows, ones_d)
        ptr_flat = (t2 + t1) * inv_sqrt_d

        pm_flat = pm_ref[bs].reshape(R, _NA)
        ptr_m = jnp.where((pm_flat > 0.5) & (kv_diag > 0.5), ptr_flat, _NEG)
        ptr_out[bs] = ptr_m.reshape(BT, _T, _NA)[:, 1:1 + _NT, :]

        cv = _dot(x2, W_cv[...])
        cv3 = cv.reshape(BT, _T, 2)
        card_out[bs] = jnp.where(cm_ref[bs] > 0.5, cv3[:, 1:9, 0:1], _NEG)
        val_out[bs] = cv3[:, 0:1, 1:2]
        return carry

    jax.lax.fori_loop(0, NB, tile, 0)


@jax.jit
def kernel(global_context, hand_cards, jokers, consumables, shop_cards,
           pack_cards, hand_mask, joker_mask, cons_mask, shop_mask, pack_mask,
           type_mask, card_mask, pointer_masks, params):
    p = params
    B = global_context.shape[0]

    ent_maskf = jnp.concatenate(
        [hand_mask, joker_mask, cons_mask, shop_mask, pack_mask],
        axis=1).astype(_BF)
    type_maskf = type_mask.astype(_BF)
    card_maskf = card_mask.astype(_BF).reshape(B, 8, 1)
    pm_pad = jnp.pad(pointer_masks.astype(_BF).transpose(0, 2, 1),
                     ((0, 0), (1, _T - 1 - _NT), (0, 0)))

    def v2(w):
        return w.reshape(1, -1)

    par = [p['W_hand'], v2(p['b_hand']), p['W_joker'], v2(p['b_joker']),
           p['W_cons'], v2(p['b_cons']), p['W_shop'], v2(p['b_shop']),
           p['W_pack'], v2(p['b_pack']), p['W_glob'], v2(p['b_glob'])]
    for i in range(_NL):
        par += [v2(p[f'l{i}_ln1_g']), v2(p[f'l{i}_ln1_b']),
                p[f'l{i}_Wq'], p[f'l{i}_Wk'], p[f'l{i}_Wv'], p[f'l{i}_Wo'],
                v2(p[f'l{i}_ln2_g']), v2(p[f'l{i}_ln2_b']),
                p[f'l{i}_W1'], v2(p[f'l{i}_b1']), p[f'l{i}_W2']]
    par += [v2(p['lnf_g']), v2(p['lnf_b']), p['W_type'], v2(p['b_type']),
            p['A_act'], p['Wq_ptr'], p['Wk_ptr'],
            jnp.stack([p['w_card'], p['w_val']], axis=1)]

    def tile_spec(shape):
        blk = (_S_TILE,) + shape[1:]
        nz = (0,) * (len(shape) - 1)
        return pl.BlockSpec(blk, lambda i, _nz=nz: (i,) + _nz)

    def full_spec(shape):
        nz = (0,) * len(shape)
        return pl.BlockSpec(shape, lambda i, _nz=nz: _nz)

    data = [global_context.astype(_BF), hand_cards.astype(_BF),
            jokers.astype(_BF), consumables.astype(_BF),
            shop_cards.astype(_BF), pack_cards.astype(_BF),
            ent_maskf, type_maskf, card_maskf, pm_pad]
    in_specs = [tile_spec(a.shape) for a in data] + \
               [full_spec(w.shape) for w in par]

    out_shape = [
        jax.ShapeDtypeStruct((B, _NA), jnp.float32),
        jax.ShapeDtypeStruct((B, _NT, _NA), jnp.float32),
        jax.ShapeDtypeStruct((B, 8, 1), jnp.float32),
        jax.ShapeDtypeStruct((B, 1, 1), jnp.float32),
    ]
    out_specs = [tile_spec(s.shape) for s in out_shape]

    type_logits, ptr_raw, card_raw, val_raw = pl.pallas_call(
        _body,
        grid=(B // _S_TILE,),
        in_specs=in_specs,
        out_specs=out_specs,
        out_shape=out_shape,
    )(*data, *par)

    return (type_logits, ptr_raw.transpose(0, 2, 1),
            card_raw.reshape(B, 8), val_raw.reshape(B))
